# baseline (speedup 1.0000x reference)
import functools

import jax
import jax.numpy as jnp
from jax import lax
from jax.experimental import pallas as pl
from jax.experimental.pallas import tpu as pltpu
from jax.experimental.pallas import tpu_sc as plsc

NC = 2
NS = 16
NW = NC * NS
NSUB = 2
GP = 8
POS_ROWS = 400
POS_Q = 8


def _do_table(wid, idx_hbm, tab, out_hbm, idxbuf, rows, sem_g, sem_s,
              rows_w, L, per_w):
    subs = [(0, 128), (128, L - 128)]
    GRP = (GP // NSUB) * L
    ngrp = per_w // GRP
    rpg = GP // NSUB

    pltpu.sync_copy(idx_hbm.at[pl.ds(wid * rows_w, rows_w)], idxbuf)

    def fire(g, h):
        for j in range(GP):
            r = g * rpg + j // NSUB
            off, sz = subs[j % NSUB]
            dst = (j // NSUB) * L + off
            pltpu.async_copy(
                tab.at[idxbuf.at[r, pl.ds(off, sz)]],
                rows.at[pl.ds(h * GRP + dst, sz)],
                sem_g,
            )

    def wait_gathers(h):
        for j in range(GP):
            off, sz = subs[j % NSUB]
            dst = (j // NSUB) * L + off
            pltpu.make_async_copy(
                tab.at[idxbuf.at[0, pl.ds(off, sz)]],
                rows.at[pl.ds(h * GRP + dst, sz)],
                sem_g,
            ).wait()

    def store(g, h):
        pltpu.async_copy(
            rows.at[pl.ds(h * GRP, GRP)],
            out_hbm.at[pl.ds(wid * per_w + g * GRP, GRP)],
            sem_s,
        )

    def wait_store(g, h):
        pltpu.make_async_copy(
            rows.at[pl.ds(h * GRP, GRP)],
            out_hbm.at[pl.ds(wid * per_w + g * GRP, GRP)],
            sem_s,
        ).wait()

    fire(0, 0)
    fire(1, 1)

    def body(g, carry):
        h = g % 2
        wait_gathers(h)
        store(g, h)
        wait_store(g, h)
        fire(g + 2, h)
        return carry

    lax.fori_loop(0, ngrp - 2, body, 0)

    for g in (ngrp - 2, ngrp - 1):
        wait_gathers(g % 2)
        store(g, g % 2)
    for g in (ngrp - 2, ngrp - 1):
        wait_store(g, g % 2)


def kernel(qids, uids, vids, clicks, Q_table, U_table, C_table, V_table, P_table):
    B, L = qids.shape
    N = B * L
    per_w = N // NW
    rows_w = B // NW
    E = Q_table.shape[1]
    CE = C_table.shape[1]
    GRP = (GP // NSUB) * L

    mesh = plsc.VectorSubcoreMesh(core_axis_name="c", subcore_axis_name="s")

    @functools.partial(
        pl.kernel,
        mesh=mesh,
        compiler_params=pltpu.CompilerParams(use_tc_tiling_on_sc=False),
        out_type=[
            jax.ShapeDtypeStruct((N, E), jnp.float32),
            jax.ShapeDtypeStruct((N, E), jnp.float32),
            jax.ShapeDtypeStruct((N, CE), jnp.float32),
            jax.ShapeDtypeStruct((N, CE), jnp.float32),
            jax.ShapeDtypeStruct((N, CE), jnp.float32),
        ],
        scratch_types=[
            pltpu.VMEM((rows_w, L), jnp.int32),
            pltpu.VMEM((2 * GRP, E), jnp.float32),
            pltpu.VMEM((2 * GRP, CE), jnp.float32),
            pltpu.VMEM((POS_ROWS, CE), jnp.float32),
            pltpu.VMEM((10, CE), jnp.float32),
            pltpu.VMEM_SHARED((2, CE), jnp.float32),
            pltpu.SemaphoreType.DMA,
            pltpu.SemaphoreType.DMA,
        ],
    )
    def k(qi_h, ui_h, vi_h, ci_h, Qt, Ut, Ct, Vt, Pt,
          oq, ou, oc, ov, opos, idxbuf, r32, r16, posb, pv, c_sh, sem_g, sem_s):
        wid = lax.axis_index("s") * NC + lax.axis_index("c")

        with jax.named_scope("q_gather"):
            _do_table(wid, qi_h, Qt, oq, idxbuf, r32, sem_g, sem_s,
                      rows_w, L, per_w)
        with jax.named_scope("u_gather"):
            _do_table(wid, ui_h, Ut, ou, idxbuf, r32, sem_g, sem_s,
                      rows_w, L, per_w)
        with jax.named_scope("v_gather"):
            _do_table(wid, vi_h, Vt, ov, idxbuf, r16, sem_g, sem_s,
                      rows_w, L, per_w)

        with jax.named_scope("c_gather"):
            sid = lax.axis_index("s")

            @pl.when(sid == 0)
            def _():
                pltpu.sync_copy(Ct, c_sh)

            plsc.subcore_barrier()
            _do_table(wid, ci_h, c_sh, oc, idxbuf, r16, sem_g, sem_s,
                      rows_w, L, per_w)

        with jax.named_scope("pos_store"):
            pltpu.sync_copy(Pt, pv)
            prow = [pv[i, :] for i in range(10)]
            for b in range(POS_ROWS // 10):
                for r in range(10):
                    posb[b * 10 + r, :] = prow[r]

            nstores = per_w // POS_ROWS

            def pos_body(t, carry):
                cps = []
                for u in range(POS_Q):
                    base = wid * per_w + (t * POS_Q + u) * POS_ROWS
                    cps.append(
                        pltpu.async_copy(
                            posb, opos.at[pl.ds(base, POS_ROWS)], sem_s
                        )
                    )
                for cp in cps:
                    cp.wait()
                return carry

            lax.fori_loop(0, nstores // POS_Q, pos_body, 0)

    oq, ou, oc, ov, opos = k(qids, uids, vids, clicks,
                             Q_table, U_table, C_table, V_table, P_table)
    return (
        oq.reshape(B, L, E) * 1.0,
        ou.reshape(B, L, E) * 1.0,
        oc.reshape(B, L, CE) * 1.0,
        ov.reshape(B, L, CE) * 1.0,
        opos.reshape(B, L, CE) * 1.0,
    )


# speedup vs baseline: 1.3843x; 1.3843x over previous
import functools

import jax
import jax.numpy as jnp
from jax import lax
from jax.experimental import pallas as pl
from jax.experimental.pallas import tpu as pltpu
from jax.experimental.pallas import tpu_sc as plsc

NC = 2
NS = 16
NW = NC * NS
NSUB = 2
GP = 8
POS_ROWS = 400
POS_Q = 8


def _do_table(wid, idx_hbm, tab, out_hbm, idxbuf, rows, sem_g, sem_s,
              rows_w, L, per_w):
    subs = [(0, 128), (128, L - 128)]
    GRP = (GP // NSUB) * L
    ngrp = per_w // GRP
    rpg = GP // NSUB

    pltpu.sync_copy(idx_hbm.at[pl.ds(wid * rows_w, rows_w)], idxbuf)

    def fire(g, h):
        for j in range(GP):
            r = g * rpg + j // NSUB
            off, sz = subs[j % NSUB]
            dst = (j // NSUB) * L + off
            pltpu.async_copy(
                tab.at[idxbuf.at[r, pl.ds(off, sz)]],
                rows.at[pl.ds(h * GRP + dst, sz)],
                sem_g,
            )

    def wait_gathers(h):
        for j in range(GP):
            off, sz = subs[j % NSUB]
            dst = (j // NSUB) * L + off
            pltpu.make_async_copy(
                tab.at[idxbuf.at[0, pl.ds(off, sz)]],
                rows.at[pl.ds(h * GRP + dst, sz)],
                sem_g,
            ).wait()

    def store(g, h):
        pltpu.async_copy(
            rows.at[pl.ds(h * GRP, GRP)],
            out_hbm.at[pl.ds(wid * per_w + g * GRP, GRP)],
            sem_s,
        )

    def wait_store(g, h):
        pltpu.make_async_copy(
            rows.at[pl.ds(h * GRP, GRP)],
            out_hbm.at[pl.ds(wid * per_w + g * GRP, GRP)],
            sem_s,
        ).wait()

    fire(0, 0)
    fire(1, 1)

    def body(g, carry):
        h = g % 2
        wait_gathers(h)
        store(g, h)
        wait_store(g, h)
        fire(g + 2, h)
        return carry

    lax.fori_loop(0, ngrp - 2, body, 0)

    for g in (ngrp - 2, ngrp - 1):
        wait_gathers(g % 2)
        store(g, g % 2)
    for g in (ngrp - 2, ngrp - 1):
        wait_store(g, g % 2)


def _transpose_tc(x):
    M, N = x.shape
    BM = 256
    BN = 640 if N % 640 == 0 else 512

    def body(x_ref, o_ref):
        o_ref[...] = x_ref[...].T

    return pl.pallas_call(
        body,
        grid=(M // BM, N // BN),
        in_specs=[pl.BlockSpec((BM, BN), lambda i, j: (i, j))],
        out_specs=pl.BlockSpec((BN, BM), lambda i, j: (j, i)),
        out_shape=jax.ShapeDtypeStruct((N, M), x.dtype),
    )(x)


def kernel(qids, uids, vids, clicks, Q_table, U_table, C_table, V_table, P_table):
    B, L = qids.shape
    N = B * L
    per_w = N // NW
    rows_w = B // NW
    E = Q_table.shape[1]
    CE = C_table.shape[1]
    GRP = (GP // NSUB) * L

    mesh = plsc.VectorSubcoreMesh(core_axis_name="c", subcore_axis_name="s")

    @functools.partial(
        pl.kernel,
        mesh=mesh,
        compiler_params=pltpu.CompilerParams(use_tc_tiling_on_sc=False),
        out_type=[
            jax.ShapeDtypeStruct((N, E), jnp.float32),
            jax.ShapeDtypeStruct((N, E), jnp.float32),
            jax.ShapeDtypeStruct((N, CE), jnp.float32),
            jax.ShapeDtypeStruct((N, CE), jnp.float32),
            jax.ShapeDtypeStruct((N, CE), jnp.float32),
        ],
        scratch_types=[
            pltpu.VMEM((rows_w, L), jnp.int32),
            pltpu.VMEM((2 * GRP, E), jnp.float32),
            pltpu.VMEM((2 * GRP, CE), jnp.float32),
            pltpu.VMEM((POS_ROWS, CE), jnp.float32),
            pltpu.VMEM((10, CE), jnp.float32),
            pltpu.VMEM_SHARED((2, CE), jnp.float32),
            pltpu.SemaphoreType.DMA,
            pltpu.SemaphoreType.DMA,
        ],
    )
    def k(qi_h, ui_h, vi_h, ci_h, Qt, Ut, Ct, Vt, Pt,
          oq, ou, oc, ov, opos, idxbuf, r32, r16, posb, pv, c_sh, sem_g, sem_s):
        wid = lax.axis_index("s") * NC + lax.axis_index("c")

        with jax.named_scope("q_gather"):
            _do_table(wid, qi_h, Qt, oq, idxbuf, r32, sem_g, sem_s,
                      rows_w, L, per_w)
        with jax.named_scope("u_gather"):
            _do_table(wid, ui_h, Ut, ou, idxbuf, r32, sem_g, sem_s,
                      rows_w, L, per_w)
        with jax.named_scope("v_gather"):
            _do_table(wid, vi_h, Vt, ov, idxbuf, r16, sem_g, sem_s,
                      rows_w, L, per_w)

        with jax.named_scope("c_gather"):
            sid = lax.axis_index("s")

            @pl.when(sid == 0)
            def _():
                pltpu.sync_copy(Ct, c_sh)

            plsc.subcore_barrier()
            _do_table(wid, ci_h, c_sh, oc, idxbuf, r16, sem_g, sem_s,
                      rows_w, L, per_w)

        with jax.named_scope("pos_store"):
            pltpu.sync_copy(Pt, pv)
            prow = [pv[i, :] for i in range(10)]
            for b in range(POS_ROWS // 10):
                for r in range(10):
                    posb[b * 10 + r, :] = prow[r]

            nstores = per_w // POS_ROWS

            def pos_body(t, carry):
                cps = []
                for u in range(POS_Q):
                    base = wid * per_w + (t * POS_Q + u) * POS_ROWS
                    cps.append(
                        pltpu.async_copy(
                            posb, opos.at[pl.ds(base, POS_ROWS)], sem_s
                        )
                    )
                for cp in cps:
                    cp.wait()
                return carry

            lax.fori_loop(0, nstores // POS_Q, pos_body, 0)

    oq, ou, oc, ov, opos = k(qids, uids, vids, clicks,
                             Q_table, U_table, C_table, V_table, P_table)

    def to_blE(flat, e):
        t = _transpose_tc(flat.reshape(B, L * e))
        return jnp.transpose(t.reshape(L, e, B), (2, 0, 1))

    return (
        to_blE(oq, E),
        to_blE(ou, E),
        to_blE(oc, CE),
        to_blE(ov, CE),
        to_blE(opos, CE),
    )


# speedup vs baseline: 1.8008x; 1.3008x over previous
import functools

import jax
import jax.numpy as jnp
from jax import lax
from jax.experimental import pallas as pl
from jax.experimental.pallas import tpu as pltpu
from jax.experimental.pallas import tpu_sc as plsc

NC = 2
NS = 16
NW = NC * NS
NSUB = 2
GP = 8


def _do_table(wid, idx_hbm, tab, out_hbm, idxbuf, rows, sem_g, sem_s,
              rows_w, L, per_w):
    subs = [(0, 128), (128, L - 128)]
    rpg = GP // NSUB
    GRP = rpg * L
    ngrp = per_w // GRP

    pltpu.sync_copy(idx_hbm.at[pl.ds(wid * rows_w, rows_w)], idxbuf)

    def fire(g, h):
        for j in range(GP):
            r = g * rpg + j // NSUB
            off, sz = subs[j % NSUB]
            dst = (j // NSUB) * L + off
            pltpu.async_copy(
                tab.at[idxbuf.at[r, pl.ds(off, sz)]],
                rows.at[pl.ds(h * GRP + dst, sz)],
                sem_g,
            )

    def wait_gathers(h):
        for j in range(GP):
            off, sz = subs[j % NSUB]
            dst = (j // NSUB) * L + off
            pltpu.make_async_copy(
                tab.at[idxbuf.at[0, pl.ds(off, sz)]],
                rows.at[pl.ds(h * GRP + dst, sz)],
                sem_g,
            ).wait()

    def store(g, h):
        pltpu.async_copy(
            rows.at[pl.ds(h * GRP, GRP)],
            out_hbm.at[pl.ds(wid * per_w + g * GRP, GRP)],
            sem_s,
        )

    def wait_store(g, h):
        pltpu.make_async_copy(
            rows.at[pl.ds(h * GRP, GRP)],
            out_hbm.at[pl.ds(wid * per_w + g * GRP, GRP)],
            sem_s,
        ).wait()

    fire(0, 0)
    fire(1, 1)

    def body(g, carry):
        h = g % 2
        wait_gathers(h)
        store(g, h)
        wait_store(g, h)
        fire(g + 2, h)
        return carry

    lax.fori_loop(0, ngrp - 2, body, 0)

    for g in (ngrp - 2, ngrp - 1):
        wait_gathers(g % 2)
        store(g, g % 2)
    for g in (ngrp - 2, ngrp - 1):
        wait_store(g, g % 2)


def _sc_gather(idx, tab):
    B, L = idx.shape
    N = B * L
    R, E = tab.shape
    per_w = N // NW
    rows_w = B // NW
    GRP = (GP // NSUB) * L
    mesh = plsc.VectorSubcoreMesh(core_axis_name="c", subcore_axis_name="s")

    @functools.partial(
        pl.kernel,
        mesh=mesh,
        compiler_params=pltpu.CompilerParams(use_tc_tiling_on_sc=False),
        out_type=jax.ShapeDtypeStruct((N, E), jnp.float32),
        scratch_types=[
            pltpu.VMEM((rows_w, L), jnp.int32),
            pltpu.VMEM((2 * GRP, E), jnp.float32),
            pltpu.SemaphoreType.DMA,
            pltpu.SemaphoreType.DMA,
        ],
    )
    def k(idx_h, tab_h, out, idxbuf, rows, sem_g, sem_s):
        wid = lax.axis_index("s") * NC + lax.axis_index("c")
        _do_table(wid, idx_h, tab_h, out, idxbuf, rows, sem_g, sem_s,
                  rows_w, L, per_w)

    return k(idx, tab)


def _transpose_tc(x):
    M, N = x.shape
    BM = 256
    BN = 640 if N % 640 == 0 else 512

    def body(x_ref, o_ref):
        o_ref[...] = x_ref[...].T

    return pl.pallas_call(
        body,
        grid=(M // BM, N // BN),
        in_specs=[pl.BlockSpec((BM, BN), lambda i, j: (i, j))],
        out_specs=pl.BlockSpec((BN, BM), lambda i, j: (j, i)),
        out_shape=jax.ShapeDtypeStruct((N, M), x.dtype),
    )(x)


def _pos_tc(P_table, B, L):
    PR, PE = P_table.shape
    rep = 320 // (PR * PE)
    pcol = jnp.tile(P_table.reshape(-1), rep).reshape(320, 1)

    def body(p_ref, o_ref):
        o_ref[...] = jnp.broadcast_to(p_ref[...], (320, B))

    return pl.pallas_call(
        body,
        grid=(L * PE // 320,),
        in_specs=[pl.BlockSpec((320, 1), lambda i: (0, 0))],
        out_specs=pl.BlockSpec((320, B), lambda i: (i, 0)),
        out_shape=jax.ShapeDtypeStruct((L * PE, B), jnp.float32),
    )(pcol)


def _click_tc(clicks, C_table):
    B, L = clicks.shape
    CE = C_table.shape[1]
    clicks_t = clicks.T
    c0 = C_table[0].reshape(CE, 1)
    c1 = C_table[1].reshape(CE, 1)
    LB = 8

    def body(cl_ref, c0_ref, c1_ref, o_ref):
        c0b = c0_ref[...]
        c1b = c1_ref[...]
        for i in range(LB):
            m = cl_ref[i:i + 1, :] == 0
            o_ref[i * CE:(i + 1) * CE, :] = jnp.where(m, c0b, c1b)

    return pl.pallas_call(
        body,
        grid=(L // LB,),
        in_specs=[
            pl.BlockSpec((LB, B), lambda i: (i, 0)),
            pl.BlockSpec((CE, 1), lambda i: (0, 0)),
            pl.BlockSpec((CE, 1), lambda i: (0, 0)),
        ],
        out_specs=pl.BlockSpec((LB * CE, B), lambda i: (i, 0)),
        out_shape=jax.ShapeDtypeStruct((L * CE, B), jnp.float32),
    )(clicks_t, c0, c1)


def kernel(qids, uids, vids, clicks, Q_table, U_table, C_table, V_table, P_table):
    B, L = qids.shape
    E = Q_table.shape[1]
    CE = C_table.shape[1]

    oq = _sc_gather(qids, Q_table)
    ou = _sc_gather(uids, U_table)
    ov = _sc_gather(vids, V_table)

    def finish(t2, e):
        return jnp.transpose(t2.reshape(L, e, B), (2, 0, 1))

    return (
        finish(_transpose_tc(oq.reshape(B, L * E)), E),
        finish(_transpose_tc(ou.reshape(B, L * E)), E),
        finish(_click_tc(clicks, C_table), CE),
        finish(_transpose_tc(ov.reshape(B, L * CE)), CE),
        finish(_pos_tc(P_table, B, L), CE),
    )


# speedup vs baseline: 2.4966x; 1.3864x over previous
import functools

import jax
import jax.numpy as jnp
from jax import lax
from jax.experimental import pallas as pl
from jax.experimental.pallas import tpu as pltpu
from jax.experimental.pallas import tpu_sc as plsc

NC = 2
NS = 16
NW = NC * NS
NSUB = 2
GP = 8


def _do_table(wid, idx_hbm, tab, out_hbm, idxbuf, rows, sem_g, sem_s,
              rows_w, L, per_w):
    subs = [(0, 128), (128, L - 128)]
    rpg = GP // NSUB
    GRP = rpg * L
    ngrp = per_w // GRP

    pltpu.sync_copy(idx_hbm.at[pl.ds(wid * rows_w, rows_w)], idxbuf)

    def fire(g, h):
        for j in range(GP):
            r = g * rpg + j // NSUB
            off, sz = subs[j % NSUB]
            dst = (j // NSUB) * L + off
            pltpu.async_copy(
                tab.at[idxbuf.at[r, pl.ds(off, sz)]],
                rows.at[pl.ds(h * GRP + dst, sz)],
                sem_g,
            )

    def wait_gathers(h):
        for j in range(GP):
            off, sz = subs[j % NSUB]
            dst = (j // NSUB) * L + off
            pltpu.make_async_copy(
                tab.at[idxbuf.at[0, pl.ds(off, sz)]],
                rows.at[pl.ds(h * GRP + dst, sz)],
                sem_g,
            ).wait()

    def store(g, h):
        pltpu.async_copy(
            rows.at[pl.ds(h * GRP, GRP)],
            out_hbm.at[pl.ds(wid * per_w + g * GRP, GRP)],
            sem_s,
        )

    def wait_store(g, h):
        pltpu.make_async_copy(
            rows.at[pl.ds(h * GRP, GRP)],
            out_hbm.at[pl.ds(wid * per_w + g * GRP, GRP)],
            sem_s,
        ).wait()

    fire(0, 0)
    fire(1, 1)

    def body(g, carry):
        h = g % 2
        wait_gathers(h)
        store(g, h)
        wait_store(g, h)
        fire(g + 2, h)
        return carry

    lax.fori_loop(0, ngrp - 2, body, 0)

    for g in (ngrp - 2, ngrp - 1):
        wait_gathers(g % 2)
        store(g, g % 2)
    for g in (ngrp - 2, ngrp - 1):
        wait_store(g, g % 2)


def _sc_gather(idx, tab):
    B, L = idx.shape
    N = B * L
    R, E = tab.shape
    per_w = N // NW
    rows_w = B // NW
    GRP = (GP // NSUB) * L
    mesh = plsc.VectorSubcoreMesh(core_axis_name="c", subcore_axis_name="s")

    @functools.partial(
        pl.kernel,
        mesh=mesh,
        compiler_params=pltpu.CompilerParams(use_tc_tiling_on_sc=False),
        out_type=jax.ShapeDtypeStruct((N, E), jnp.float32),
        scratch_types=[
            pltpu.VMEM((rows_w, L), jnp.int32),
            pltpu.VMEM((2 * GRP, E), jnp.float32),
            pltpu.SemaphoreType.DMA,
            pltpu.SemaphoreType.DMA,
        ],
    )
    def k(idx_h, tab_h, out, idxbuf, rows, sem_g, sem_s):
        wid = lax.axis_index("s") * NC + lax.axis_index("c")
        _do_table(wid, idx_h, tab_h, out, idxbuf, rows, sem_g, sem_s,
                  rows_w, L, per_w)

    return k(idx, tab)


def _transpose_flat_tc(flat, B, LE):
    BM = 256
    S = LE // 128
    x2 = flat.reshape(B * S, 128)

    def body(x_ref, o_ref):
        x3 = x_ref[...].reshape(BM, S, 128)
        for j in range(S):
            o_ref[pl.ds(j * 128, 128), :] = x3[:, j, :].T

    return pl.pallas_call(
        body,
        grid=(B // BM,),
        in_specs=[pl.BlockSpec((BM * S, 128), lambda i: (i, 0))],
        out_specs=pl.BlockSpec((LE, BM), lambda i: (0, i)),
        out_shape=jax.ShapeDtypeStruct((LE, B), flat.dtype),
    )(x2)


def _pos_tc(P_table, B, L):
    PR, PE = P_table.shape
    rep = 320 // (PR * PE)
    pcol = jnp.tile(P_table.reshape(-1), rep).reshape(320, 1)

    def body(p_ref, o_ref):
        o_ref[...] = jnp.broadcast_to(p_ref[...], (320, B))

    return pl.pallas_call(
        body,
        grid=(L * PE // 320,),
        in_specs=[pl.BlockSpec((320, 1), lambda i: (0, 0))],
        out_specs=pl.BlockSpec((320, B), lambda i: (i, 0)),
        out_shape=jax.ShapeDtypeStruct((L * PE, B), jnp.float32),
    )(pcol)


def _click_tc(clicks, C_table):
    B, L = clicks.shape
    CE = C_table.shape[1]
    clicks_t = clicks.T
    c0 = C_table[0].reshape(CE, 1)
    c1 = C_table[1].reshape(CE, 1)
    LB = 8

    def body(cl_ref, c0_ref, c1_ref, o_ref):
        c0b = c0_ref[...]
        c1b = c1_ref[...]
        for i in range(LB):
            m = cl_ref[i:i + 1, :] == 0
            o_ref[i * CE:(i + 1) * CE, :] = jnp.where(m, c0b, c1b)

    return pl.pallas_call(
        body,
        grid=(L // LB,),
        in_specs=[
            pl.BlockSpec((LB, B), lambda i: (i, 0)),
            pl.BlockSpec((CE, 1), lambda i: (0, 0)),
            pl.BlockSpec((CE, 1), lambda i: (0, 0)),
        ],
        out_specs=pl.BlockSpec((LB * CE, B), lambda i: (i, 0)),
        out_shape=jax.ShapeDtypeStruct((L * CE, B), jnp.float32),
    )(clicks_t, c0, c1)


def kernel(qids, uids, vids, clicks, Q_table, U_table, C_table, V_table, P_table):
    B, L = qids.shape
    E = Q_table.shape[1]
    CE = C_table.shape[1]

    oq = _sc_gather(qids, Q_table)
    ou = _sc_gather(uids, U_table)
    ov = _sc_gather(vids, V_table)

    def finish(t2, e):
        return jnp.transpose(t2.reshape(L, e, B), (2, 0, 1))

    return (
        finish(_transpose_flat_tc(oq, B, L * E), E),
        finish(_transpose_flat_tc(ou, B, L * E), E),
        finish(_click_tc(clicks, C_table), CE),
        finish(_transpose_flat_tc(ov, B, L * CE), CE),
        finish(_pos_tc(P_table, B, L), CE),
    )
